# baseline (speedup 1.0000x reference)
import functools

import jax
import jax.numpy as jnp
from jax import lax
from jax.experimental import pallas as pl
from jax.experimental.pallas import tpu as pltpu
from jax.experimental.pallas import tpu_sc as plsc

NC, NS, LANES = 2, 16, 16
NW = NC * NS


def _sc_mesh():
    return plsc.VectorSubcoreMesh(
        core_axis_name="c", subcore_axis_name="s", num_cores=NC, num_subcores=NS
    )


def _wid():
    return lax.axis_index("s") * NC + lax.axis_index("c")


def _scores_body(h_ref, w_ref, b_ref, scores_ref, hs_ref):
    hv = h_ref[...]
    w = w_ref[...]
    wt_full = lax.dot_general(hv, w, (((1,), (1,)), ((), ())))
    wt = wt_full[:, 0:1]
    s = jax.nn.sigmoid(wt + b_ref[0])
    scores_ref[...] = s
    hs_ref[...] = hv * s


def _scores_tc(h, W2, b1):
    N, D = h.shape
    return pl.pallas_call(
        _scores_body,
        in_specs=[
            pl.BlockSpec(memory_space=pltpu.MemorySpace.VMEM),
            pl.BlockSpec(memory_space=pltpu.MemorySpace.VMEM),
            pl.BlockSpec(memory_space=pltpu.MemorySpace.SMEM),
        ],
        out_shape=[
            jax.ShapeDtypeStruct((N, 1), jnp.float32),
            jax.ShapeDtypeStruct((N, D), jnp.float32),
        ],
    )(h, W2, b1)


_BR = 256


def _rank_body(sc_ref, sr_ref, rank_ref):
    i = pl.program_id(0)
    sc = sc_ref[...]
    sr = sr_ref[...]
    n = sr.shape[1]
    gt = (sr > sc).astype(jnp.float32)
    jrow = lax.broadcasted_iota(jnp.int32, (_BR, n), 1)
    irow = lax.broadcasted_iota(jnp.int32, (_BR, n), 0) + i * _BR
    tie = ((sr == sc) & (jrow < irow)).astype(jnp.float32)
    cnt = jnp.sum(gt + tie, axis=1, keepdims=True)
    rank_ref[...] = cnt.astype(jnp.int32)


def _rank_tc(s_col, s_row):
    N = s_col.shape[0]
    return pl.pallas_call(
        _rank_body,
        grid=(N // _BR,),
        in_specs=[
            pl.BlockSpec((_BR, 1), lambda i: (i, 0)),
            pl.BlockSpec((1, N), lambda i: (0, 0)),
        ],
        out_specs=pl.BlockSpec((_BR, 1), lambda i: (i, 0)),
        out_shape=jax.ShapeDtypeStruct((N, 1), jnp.int32),
    )(s_col, s_row)


def _select_sc(rank1, kk):
    N = rank1.shape[0]
    per = kk // NW

    @functools.partial(
        pl.kernel,
        out_type=jax.ShapeDtypeStruct((kk,), jnp.int32),
        mesh=_sc_mesh(),
        compiler_params=pltpu.CompilerParams(needs_layout_passes=False),
        scratch_types=[
            pltpu.VMEM((N,), jnp.int32),
            pltpu.VMEM((per,), jnp.int32),
        ],
    )
    def sel(rank_hbm, idx_hbm, rank_v, buf_v):
        lo = _wid() * per
        pltpu.sync_copy(rank_hbm, rank_v)

        def body(c, carry):
            r = rank_v[pl.ds(c * LANES, LANES)]
            iv = lax.iota(jnp.int32, LANES) + c * LANES
            m = (r >= lo) & (r < lo + per)
            rr = jnp.where(m, r - lo, 0)
            plsc.store_scatter(buf_v, [rr], iv, mask=m)
            return carry

        lax.fori_loop(0, N // LANES, body, 0)
        pltpu.sync_copy(buf_v, idx_hbm.at[pl.ds(lo, per)])

    return sel(rank1)


def _gather_sc(g, hs, idx):
    N = g.shape[0]
    D = hs.shape[1]
    kk = idx.shape[0]
    per = kk // NW
    chunks = per // LANES

    @functools.partial(
        pl.kernel,
        out_type=[
            jax.ShapeDtypeStruct((kk, N), jnp.float32),
            jax.ShapeDtypeStruct((kk, D), jnp.float32),
        ],
        mesh=_sc_mesh(),
        compiler_params=pltpu.CompilerParams(needs_layout_passes=False),
        scratch_types=[
            pltpu.VMEM((per,), jnp.int32),
            pltpu.VMEM((LANES, N), jnp.float32),
            pltpu.VMEM((per, D), jnp.float32),
            pltpu.SemaphoreType.DMA,
            pltpu.SemaphoreType.DMA,
        ],
    )
    def gat(g_hbm, hs_hbm, idx_hbm, a_hbm, nh_hbm, idx_v, grow_v, hrow_v, gsem, hsem):
        base = _wid() * per
        pltpu.sync_copy(idx_hbm.at[pl.ds(base, per)], idx_v)
        hcp = pltpu.async_copy(hs_hbm.at[idx_v], hrow_v, hsem)
        for t in range(chunks):
            ivr = idx_v[pl.ds(t * LANES, LANES)]
            pltpu.async_copy(g_hbm.at[ivr], grow_v, gsem).wait()
            pltpu.sync_copy(grow_v, a_hbm.at[pl.ds(base + t * LANES, LANES)])
        hcp.wait()
        pltpu.sync_copy(hrow_v, nh_hbm.at[pl.ds(base, per)])

    return gat(g, hs, idx)


_BM, _BK, _BN = 512, 512, 1024


def _mm_body(a_ref, g_ref, o_ref):
    k = pl.program_id(2)

    @pl.when(k == 0)
    def _():
        o_ref[...] = jnp.zeros_like(o_ref)

    a = a_ref[...].astype(jnp.bfloat16)
    gb = g_ref[...].astype(jnp.bfloat16)
    o_ref[...] += jnp.dot(a, gb, preferred_element_type=jnp.float32)


def _matmul_tc(A, g):
    kk, N = A.shape
    return pl.pallas_call(
        _mm_body,
        grid=(kk // _BM, N // _BN, N // _BK),
        in_specs=[
            pl.BlockSpec((_BM, _BK), lambda i, j, k: (i, k)),
            pl.BlockSpec((_BK, _BN), lambda i, j, k: (k, j)),
        ],
        out_specs=pl.BlockSpec((_BM, _BN), lambda i, j, k: (i, j)),
        out_shape=jax.ShapeDtypeStruct((kk, N), jnp.float32),
    )(A, g)


def _colsel_sc(B1, idx, N):
    kk = idx.shape[0]
    per = kk // NW

    @functools.partial(
        pl.kernel,
        out_type=jax.ShapeDtypeStruct((kk * kk,), jnp.float32),
        mesh=_sc_mesh(),
        compiler_params=pltpu.CompilerParams(needs_layout_passes=False),
        scratch_types=[
            pltpu.VMEM((kk,), jnp.int32),
            pltpu.VMEM((N,), jnp.float32),
            pltpu.VMEM((kk,), jnp.float32),
        ],
    )
    def cs(b_hbm, idx_hbm, out_hbm, idx_v, row_v, orow_v):
        base = _wid() * per
        pltpu.sync_copy(idx_hbm, idx_v)

        def rbody(r, carry):
            row = base + r
            pltpu.sync_copy(b_hbm.at[pl.ds(row * N, N)], row_v)
            for c in range(kk // LANES):
                ii = idx_v[pl.ds(c * LANES, LANES)]
                vals = plsc.load_gather(row_v, [ii])
                orow_v[pl.ds(c * LANES, LANES)] = jnp.where(vals != 0.0, 1.0, 0.0)
            pltpu.sync_copy(orow_v, out_hbm.at[pl.ds(row * kk, kk)])
            return carry

        lax.fori_loop(0, per, rbody, 0)

    return cs(B1, idx)


def kernel(g, h, ep, W, b):
    N, D = h.shape
    kk = max(2, N // 2)
    Wp = jnp.pad(W, ((0, 127), (0, 0)))
    scores, hs = _scores_tc(h, Wp, b)
    rank = _rank_tc(scores, scores.reshape(1, N))
    idx = _select_sc(rank.reshape(N), kk)
    A, new_h = _gather_sc(g, hs, idx)
    B = _matmul_tc(A, g)
    un_g = _colsel_sc(B.reshape(-1), idx, N).reshape(kk, kk)
    return un_g, new_h, idx


# speedup vs baseline: 1.4621x; 1.4621x over previous
import functools

import jax
import jax.numpy as jnp
from jax import lax
from jax.experimental import pallas as pl
from jax.experimental.pallas import tpu as pltpu
from jax.experimental.pallas import tpu_sc as plsc

NC, NS, LANES = 2, 16, 16
NW = NC * NS


def _sc_mesh():
    return plsc.VectorSubcoreMesh(
        core_axis_name="c", subcore_axis_name="s", num_cores=NC, num_subcores=NS
    )


def _wid():
    return lax.axis_index("s") * NC + lax.axis_index("c")


def _scores_body(h_ref, w_ref, b_ref, scores_ref, hs_ref):
    hv = h_ref[...]
    w = w_ref[...]
    wt_full = lax.dot_general(hv, w, (((1,), (1,)), ((), ())))
    wt = wt_full[:, 0:1]
    s = jax.nn.sigmoid(wt + b_ref[0])
    scores_ref[...] = s
    hs_ref[...] = hv * s


def _scores_tc(h, W2, b1):
    N, D = h.shape
    return pl.pallas_call(
        _scores_body,
        in_specs=[
            pl.BlockSpec(memory_space=pltpu.MemorySpace.VMEM),
            pl.BlockSpec(memory_space=pltpu.MemorySpace.VMEM),
            pl.BlockSpec(memory_space=pltpu.MemorySpace.SMEM),
        ],
        out_shape=[
            jax.ShapeDtypeStruct((N, 1), jnp.float32),
            jax.ShapeDtypeStruct((N, D), jnp.float32),
        ],
    )(h, W2, b1)


_BR = 256


def _rank_body(sc_ref, sr_ref, rank_ref):
    i = pl.program_id(0)
    sc = sc_ref[...]
    sr = sr_ref[...]
    n = sr.shape[1]
    gt = (sr > sc).astype(jnp.float32)
    jrow = lax.broadcasted_iota(jnp.int32, (_BR, n), 1)
    irow = lax.broadcasted_iota(jnp.int32, (_BR, n), 0) + i * _BR
    tie = ((sr == sc) & (jrow < irow)).astype(jnp.float32)
    cnt = jnp.sum(gt + tie, axis=1, keepdims=True)
    rank_ref[...] = cnt.astype(jnp.int32)


def _rank_tc(s_col, s_row):
    N = s_col.shape[0]
    return pl.pallas_call(
        _rank_body,
        grid=(N // _BR,),
        in_specs=[
            pl.BlockSpec((_BR, 1), lambda i: (i, 0)),
            pl.BlockSpec((1, N), lambda i: (0, 0)),
        ],
        out_specs=pl.BlockSpec((_BR, 1), lambda i: (i, 0)),
        out_shape=jax.ShapeDtypeStruct((N, 1), jnp.int32),
    )(s_col, s_row)


def _select_sc(rank1, kk):
    N = rank1.shape[0]
    per = kk // NW

    @functools.partial(
        pl.kernel,
        out_type=jax.ShapeDtypeStruct((kk,), jnp.int32),
        mesh=_sc_mesh(),
        compiler_params=pltpu.CompilerParams(needs_layout_passes=False),
        scratch_types=[
            pltpu.VMEM((N,), jnp.int32),
            pltpu.VMEM((per,), jnp.int32),
        ],
    )
    def sel(rank_hbm, idx_hbm, rank_v, buf_v):
        lo = _wid() * per
        pltpu.sync_copy(rank_hbm, rank_v)

        def body(c, carry):
            r = rank_v[pl.ds(c * LANES, LANES)]
            iv = lax.iota(jnp.int32, LANES) + c * LANES
            m = (r >= lo) & (r < lo + per)
            rr = jnp.where(m, r - lo, 0)
            plsc.store_scatter(buf_v, [rr], iv, mask=m)
            return carry

        lax.fori_loop(0, N // LANES, body, 0)
        pltpu.sync_copy(buf_v, idx_hbm.at[pl.ds(lo, per)])

    return sel(rank1)


def _gather_sc(g, hs, idx):
    N = g.shape[0]
    D = hs.shape[1]
    kk = idx.shape[0]
    per = kk // NW
    chunks = per // LANES

    @functools.partial(
        pl.kernel,
        out_type=[
            jax.ShapeDtypeStruct((kk, N), jnp.float32),
            jax.ShapeDtypeStruct((kk, D), jnp.float32),
        ],
        mesh=_sc_mesh(),
        compiler_params=pltpu.CompilerParams(needs_layout_passes=False),
        scratch_types=[
            pltpu.VMEM((per,), jnp.int32),
            pltpu.VMEM((LANES, N), jnp.float32),
            pltpu.VMEM((per, D), jnp.float32),
            pltpu.SemaphoreType.DMA,
            pltpu.SemaphoreType.DMA,
        ],
    )
    def gat(g_hbm, hs_hbm, idx_hbm, a_hbm, nh_hbm, idx_v, grow_v, hrow_v, gsem, hsem):
        base = _wid() * per
        pltpu.sync_copy(idx_hbm.at[pl.ds(base, per)], idx_v)
        hcp = pltpu.async_copy(hs_hbm.at[idx_v], hrow_v, hsem)
        for t in range(chunks):
            ivr = idx_v[pl.ds(t * LANES, LANES)]
            pltpu.async_copy(g_hbm.at[ivr], grow_v, gsem).wait()
            pltpu.sync_copy(grow_v, a_hbm.at[pl.ds(base + t * LANES, LANES)])
        hcp.wait()
        pltpu.sync_copy(hrow_v, nh_hbm.at[pl.ds(base, per)])

    return gat(g, hs, idx)


_BK = 256


def _mm_body(a_ref, g_ref, o_ref):
    k = pl.program_id(0)

    @pl.when(k == 0)
    def _():
        o_ref[...] = jnp.zeros_like(o_ref)

    o_ref[...] += jnp.dot(a_ref[...], g_ref[...], preferred_element_type=jnp.float32)


def _matmul_tc(A, g):
    kk, N = A.shape
    return pl.pallas_call(
        _mm_body,
        grid=(N // _BK,),
        in_specs=[
            pl.BlockSpec((kk, _BK), lambda k: (0, k)),
            pl.BlockSpec((_BK, N), lambda k: (k, 0)),
        ],
        out_specs=pl.BlockSpec((kk, N), lambda k: (0, 0)),
        out_shape=jax.ShapeDtypeStruct((kk, N), jnp.float32),
    )(A, g)


_RG = 8


def _colsel_sc(B, idx):
    kk, N = B.shape
    per = kk // NW
    ngrp = per // _RG

    @functools.partial(
        pl.kernel,
        out_type=jax.ShapeDtypeStruct((kk, kk), jnp.float32),
        mesh=_sc_mesh(),
        compiler_params=pltpu.CompilerParams(needs_layout_passes=False),
        scratch_types=[
            pltpu.VMEM((kk,), jnp.int32),
            pltpu.VMEM((2, _RG, N), jnp.float32),
            pltpu.VMEM((_RG, kk), jnp.float32),
            pltpu.SemaphoreType.DMA,
            pltpu.SemaphoreType.DMA,
        ],
    )
    def cs(b_hbm, idx_hbm, out_hbm, idx_v, rows_v, out_v, sem0, sem1):
        base = _wid() * per
        pltpu.sync_copy(idx_hbm, idx_v)
        sems = (sem0, sem1)
        pltpu.async_copy(b_hbm.at[pl.ds(base, _RG)], rows_v.at[0], sems[0])

        def sbody(s, carry):
            for bi in range(2):
                gidx = s * 2 + bi
                row0 = base + gidx * _RG
                pltpu.make_async_copy(
                    b_hbm.at[pl.ds(row0, _RG)], rows_v.at[bi], sems[bi]
                ).wait()

                @pl.when(gidx + 1 < ngrp)
                def _():
                    pltpu.async_copy(
                        b_hbm.at[pl.ds(row0 + _RG, _RG)],
                        rows_v.at[1 - bi],
                        sems[1 - bi],
                    )

                for rloc in range(_RG):
                    bvec = jnp.full((LANES,), bi, jnp.int32)
                    rvec = jnp.full((LANES,), rloc, jnp.int32)

                    def cbody(c, carry2):
                        ii = idx_v[pl.ds(c * LANES, LANES)]
                        vals = plsc.load_gather(rows_v, [bvec, rvec, ii])
                        res = jnp.where(vals != 0.0, 1.0, 0.0)
                        pv = lax.iota(jnp.int32, LANES) + c * LANES
                        plsc.store_scatter(out_v, [rvec, pv], res)
                        return carry2

                    lax.fori_loop(0, kk // LANES, cbody, 0, unroll=8)
                pltpu.sync_copy(out_v, out_hbm.at[pl.ds(row0, _RG)])
            return carry

        lax.fori_loop(0, ngrp // 2, sbody, 0)

    return cs(B, idx)


def kernel(g, h, ep, W, b):
    N, D = h.shape
    kk = max(2, N // 2)
    Wp = jnp.pad(W, ((0, 127), (0, 0)))
    scores, hs = _scores_tc(h, Wp, b)
    rank = _rank_tc(scores, scores.reshape(1, N))
    idx = _select_sc(rank.reshape(N), kk)
    A, new_h = _gather_sc(g, hs, idx)
    B = _matmul_tc(A, g)
    un_g = _colsel_sc(B, idx)
    return un_g, new_h, idx
